# Initial kernel scaffold; baseline (speedup 1.0000x reference)
import jax
import jax.numpy as jnp
from jax.experimental import pallas as pl


def kernel(grid_inte, grid, feat, lbs, wei):
    raise NotImplementedError("write your pallas kernel here")


# speedup vs baseline: 1.8883x; 1.8883x over previous
import functools

import jax
import jax.numpy as jnp
from jax import lax
from jax.experimental import pallas as pl
from jax.experimental.pallas import tpu as pltpu
from jax.experimental.pallas import tpu_sc as plsc

_QC = 8


def _mm_rows(x_ref, u_ref, y_ref):
    y_ref[0] = jnp.dot(u_ref[...], x_ref[0], preferred_element_type=jnp.float32)


def _mm_cols(y_ref, u_ref, o_ref):
    o_ref[...] = jnp.dot(u_ref[...], y_ref[0], preferred_element_type=jnp.float32)


def kernel(grid_inte, grid, feat, lbs, wei):
    f32 = jnp.float32
    N, D = feat.shape[0], feat.shape[1]
    r, c = grid.shape[1], grid.shape[2]
    r_in, c_in = grid_inte.shape[1], grid_inte.shape[2]
    ncls = lbs.shape[1]
    Q = r_in * c_in
    RC = r * c
    NQ = N * Q

    gX = grid[0, 2, 2, 0] - grid[0, 1, 2, 0]
    gY = grid[0, 2, 2, 1] - grid[0, 2, 1, 1]
    rp_x = jnp.floor(grid_inte[0, :, :, 0] / gX).reshape(-1)
    rp_y = jnp.floor(grid_inte[0, :, :, 1] / gY).reshape(-1)
    locs = []
    for i in range(-2, 3):
        for j in range(-2, 3):
            px = jnp.clip(rp_x + i, 0, r - 1)
            py = jnp.clip(rp_y + j, 0, c - 1)
            locs.append((py * r + px).astype(jnp.int32))
    loc_all = jnp.stack(locs, 0)
    idxg = (jnp.arange(N, dtype=jnp.int32)[None, :, None] * RC
            + loc_all[:, None, :])
    idxr = idxg.transpose(1, 2, 0).reshape(NQ, 25)
    idxp = jnp.pad(idxr, ((0, 0), (0, 32 - 25)))
    nch = NQ // _QC
    idx_blocked = idxp.reshape(nch, _QC, 32).transpose(0, 2, 1)
    idx_blocked = jnp.clip(idx_blocked.reshape(nch, 32 * _QC), 0, N * RC - 1)

    U = jax.image.resize(jnp.eye(c_in, dtype=f32), (c, c_in), method="bilinear")
    finy = feat.transpose(0, 3, 2, 1).reshape(N, c_in, r_in * D)
    Y = pl.pallas_call(
        _mm_rows,
        grid=(N,),
        in_specs=[
            pl.BlockSpec((1, c_in, r_in * D), lambda n: (n, 0, 0)),
            pl.BlockSpec((c, c_in), lambda n: (0, 0)),
        ],
        out_specs=pl.BlockSpec((1, c, r_in * D), lambda n: (n, 0, 0)),
        out_shape=jax.ShapeDtypeStruct((N, c, r_in * D), f32),
    )(finy, U)
    Y2 = Y.reshape(N * c, r_in, D)
    fgT = pl.pallas_call(
        _mm_cols,
        grid=(N * c,),
        in_specs=[
            pl.BlockSpec((1, r_in, D), lambda g: (g, 0, 0)),
            pl.BlockSpec((r, r_in), lambda g: (0, 0)),
        ],
        out_specs=pl.BlockSpec((r, D), lambda g: (g, 0)),
        out_shape=jax.ShapeDtypeStruct((N * c * r, D), f32),
    )(Y2, U)

    featqT = feat.reshape(N, D, Q).transpose(0, 2, 1).reshape(NQ, D)
    lbsT = lbs.transpose(0, 3, 2, 1).reshape(N * RC, ncls)
    lbsTp = jnp.pad(lbsT, ((0, 0), (0, 32 - ncls)))
    wsc = jnp.broadcast_to((wei[0] * (1.0 / D)).astype(f32), (16,))

    info = plsc.get_sparse_core_info()
    nw = info.num_cores * info.num_subcores
    cpw = nch // nw
    nd16 = D // 16
    mesh = plsc.VectorSubcoreMesh(core_axis_name="c", subcore_axis_name="s")

    @functools.partial(
        pl.kernel,
        out_type=jax.ShapeDtypeStruct((NQ, 32), f32),
        mesh=mesh,
        scratch_types=[
            pltpu.VMEM((32 * _QC,), jnp.int32),
            pltpu.VMEM((_QC, D), f32),
            pltpu.VMEM((25 * _QC, D), f32),
            pltpu.VMEM((25 * _QC, 32), f32),
            pltpu.VMEM((_QC, 32), f32),
            pltpu.VMEM((16,), f32),
            pltpu.SemaphoreType.DMA,
        ],
        compiler_params=pltpu.CompilerParams(use_tc_tiling_on_sc=False),
    )
    def _sc(fgT_hbm, fq_hbm, lbs_hbm, idx_hbm, wsc_hbm, out_hbm,
            idx_v, fq_v, frows_v, lrows_v, out_v, wsc_v, sem):
        w = lax.axis_index("s") * info.num_cores + lax.axis_index("c")
        pltpu.sync_copy(wsc_hbm, wsc_v)
        wreg = wsc_v[...]
        lane = lax.iota(jnp.int32, 16)
        perms = [(lane + k) & 15 for k in (8, 4, 2, 1)]

        def chunk_body(ch, _):
            gch = w * cpw + ch
            pltpu.sync_copy(idx_hbm.at[gch], idx_v)
            pltpu.sync_copy(fq_hbm.at[pl.ds(gch * _QC, _QC)], fq_v)
            d1 = pltpu.async_copy(fgT_hbm.at[idx_v.at[pl.ds(0, 96)]],
                                  frows_v.at[pl.ds(0, 96)], sem)
            d2 = pltpu.async_copy(fgT_hbm.at[idx_v.at[pl.ds(96, 104)]],
                                  frows_v.at[pl.ds(96, 104)], sem)
            d3 = pltpu.async_copy(lbs_hbm.at[idx_v.at[pl.ds(0, 96)]],
                                  lrows_v.at[pl.ds(0, 96)], sem)
            d4 = pltpu.async_copy(lbs_hbm.at[idx_v.at[pl.ds(96, 104)]],
                                  lrows_v.at[pl.ds(96, 104)], sem)
            d1.wait(); d2.wait(); d3.wait(); d4.wait()

            def q_body(qi, _):
                fqc = [fq_v[qi, pl.ds(16 * t, 16)] for t in range(nd16)]

                def o_body(o, carry):
                    a0, a1, dsum = carry
                    row = o * _QC + qi
                    s = jnp.zeros((16,), f32)
                    for t in range(nd16):
                        dd = frows_v[row, pl.ds(16 * t, 16)] - fqc[t]
                        s = s + dd * dd
                    for p in perms:
                        s = s + lax.gather(
                            s, p[:, None],
                            lax.GatherDimensionNumbers(
                                offset_dims=(), collapsed_slice_dims=(0,),
                                start_index_map=(0,)),
                            slice_sizes=(1,),
                            mode=lax.GatherScatterMode.PROMISE_IN_BOUNDS)
                    e = jnp.exp(-(s * wreg))
                    l0 = lrows_v[row, pl.ds(0, 16)]
                    l1 = lrows_v[row, pl.ds(16, 16)]
                    return (a0 + l0 * e, a1 + l1 * e, dsum + e)

                z = jnp.zeros((16,), f32)
                a0, a1, dsum = lax.fori_loop(0, 25, o_body, (z, z, z))
                dm = jnp.maximum(dsum, 1e-15)
                out_v[qi, pl.ds(0, 16)] = a0 / dm
                out_v[qi, pl.ds(16, 16)] = a1 / dm
                return 0

            lax.fori_loop(0, _QC, q_body, 0)
            pltpu.sync_copy(out_v, out_hbm.at[pl.ds(gch * _QC, _QC)])
            return 0

        lax.fori_loop(0, cpw, chunk_body, 0)

    out = _sc(fgT, featqT, lbsTp, idx_blocked, wsc)
    return (out.reshape(N, Q, 32)[:, :, :ncls]
            .transpose(0, 2, 1).reshape(N, ncls, r_in, c_in))


# speedup vs baseline: 3.2018x; 1.6956x over previous
import functools

import jax
import jax.numpy as jnp
import numpy as np
from jax import lax
from jax.experimental import pallas as pl
from jax.experimental.pallas import tpu as pltpu
from jax.experimental.pallas import tpu_sc as plsc

_QC = 8


def _mm_rows(x_ref, u_ref, y_ref):
    y_ref[0] = jnp.dot(u_ref[...], x_ref[0], preferred_element_type=jnp.float32)


def _make_mm_cols(D, gpb):
    def _mm_cols(y_ref, u_ref, o_ref):
        cat = jnp.concatenate([y_ref[0, j] for j in range(gpb)], axis=1)
        res = jnp.dot(u_ref[...], cat, preferred_element_type=jnp.float32)
        for j in range(gpb):
            rj = res[:, j * D:(j + 1) * D]
            lo = rj[:, :D // 2].astype(jnp.bfloat16)
            hi = rj[:, D // 2:].astype(jnp.bfloat16)
            lo32 = jax.lax.bitcast_convert_type(lo, jnp.uint16).astype(jnp.int32)
            hi32 = jax.lax.bitcast_convert_type(hi, jnp.uint16).astype(jnp.int32)
            o_ref[0, j] = lo32 | (hi32 << jnp.int32(16))

    return _mm_cols


def kernel(grid_inte, grid, feat, lbs, wei):
    f32 = jnp.float32
    N, D = feat.shape[0], feat.shape[1]
    r, c = grid.shape[1], grid.shape[2]
    r_in, c_in = grid_inte.shape[1], grid_inte.shape[2]
    ncls = lbs.shape[1]
    Q = r_in * c_in
    RC = r * c
    NQ = N * Q

    gX = grid[0, 2, 2, 0] - grid[0, 1, 2, 0]
    gY = grid[0, 2, 2, 1] - grid[0, 2, 1, 1]
    rp_x = jnp.floor(grid_inte[0, :, :, 0] / gX).reshape(-1)
    rp_y = jnp.floor(grid_inte[0, :, :, 1] / gY).reshape(-1)
    locs = []
    for i in range(-2, 3):
        for j in range(-2, 3):
            px = jnp.clip(rp_x + i, 0, r - 1)
            py = jnp.clip(rp_y + j, 0, c - 1)
            locs.append((px * c + py).astype(jnp.int32))
    loc_all = jnp.stack(locs, 0)
    idxg = (jnp.arange(N, dtype=jnp.int32)[None, :, None] * RC
            + loc_all[:, None, :])
    idxr = idxg.transpose(1, 2, 0).reshape(NQ, 25)
    idxp = jnp.pad(idxr, ((0, 0), (0, 32 - 25)))
    nch = NQ // _QC
    idx_blocked = idxp.reshape(nch, _QC, 32).transpose(0, 2, 1)
    idx_blocked = jnp.clip(idx_blocked.reshape(nch * 32 * _QC), 0, N * RC - 1)

    U = jax.image.resize(jnp.eye(c_in, dtype=f32), (c, c_in), method="bilinear")
    featT = feat.transpose(0, 2, 3, 1)
    Y = pl.pallas_call(
        _mm_rows,
        grid=(N,),
        in_specs=[
            pl.BlockSpec((1, r_in, c_in * D), lambda n: (n, 0, 0)),
            pl.BlockSpec((r, r_in), lambda n: (0, 0)),
        ],
        out_specs=pl.BlockSpec((1, r, c_in * D), lambda n: (n, 0, 0)),
        out_shape=jax.ShapeDtypeStruct((N, r, c_in * D), f32),
    )(featT.reshape(N, r_in, c_in * D), U)
    gpb = 8
    Y2 = Y.reshape(N * r // gpb, gpb, c_in, D)
    fgT = pl.pallas_call(
        _make_mm_cols(D, gpb),
        grid=(N * r // gpb,),
        in_specs=[
            pl.BlockSpec((1, gpb, c_in, D), lambda g: (g, 0, 0, 0)),
            pl.BlockSpec((c, c_in), lambda g: (0, 0)),
        ],
        out_specs=pl.BlockSpec((1, gpb, c, D // 2), lambda g: (g, 0, 0, 0)),
        out_shape=jax.ShapeDtypeStruct((N * r // gpb, gpb, c, D // 2), jnp.int32),
    )(Y2, U)
    fgT3 = fgT.reshape(N * r * c, D // 2)

    featqT = featT.reshape(NQ, D)
    lbsT = lbs.transpose(0, 2, 3, 1).reshape(N * RC, ncls)
    lbsTp = jnp.pad(lbsT, ((0, 0), (0, 32 - ncls)))
    wsc = jnp.broadcast_to((wei[0] * (1.0 / D)).astype(f32), (16,))

    info = plsc.get_sparse_core_info()
    nw = info.num_cores * info.num_subcores
    cpw = nch // nw
    qpw = cpw * _QC
    rows = 25 * _QC
    nb32 = D // 32
    mesh = plsc.VectorSubcoreMesh(core_axis_name="c", subcore_axis_name="s")

    @functools.partial(
        pl.kernel,
        out_type=jax.ShapeDtypeStruct((NQ, 32), f32),
        mesh=mesh,
        scratch_types=[
            pltpu.VMEM((cpw * 32 * _QC,), jnp.int32),
            pltpu.VMEM((qpw, 32), f32),
            pltpu.VMEM((16,), f32),
        ]
        + 2 * [
            pltpu.VMEM((_QC, D), f32),
            pltpu.VMEM((rows, D // 2), jnp.int32),
            pltpu.VMEM((rows, 32), f32),
            pltpu.SemaphoreType.DMA,
        ],
        compiler_params=pltpu.CompilerParams(use_tc_tiling_on_sc=False),
    )
    def _sc(fgT_hbm, fq_hbm, lbs_hbm, idx_hbm, wsc_hbm, out_hbm,
            idx_v, outb_v, wsc_v,
            fq_a, fr_a, lr_a, sem_a, fq_b, fr_b, lr_b, sem_b):
        w = lax.axis_index("s") * info.num_cores + lax.axis_index("c")
        pltpu.sync_copy(wsc_hbm, wsc_v)
        pltpu.sync_copy(idx_hbm.at[pl.ds(w * cpw * 256, cpw * 256)], idx_v)
        wreg = wsc_v[...]
        lane = lax.iota(jnp.int32, 16)
        perms = [(lane + k) & 15 for k in (8, 4, 2, 1)]

        def fire(ch, fqb, frb, lrb, sem):
            i0 = idx_v.at[pl.ds(ch * 256, 96)]
            i1 = idx_v.at[pl.ds(ch * 256 + 96, 104)]
            pltpu.async_copy(
                fq_hbm.at[pl.ds((w * cpw + ch) * _QC, _QC)], fqb, sem)
            pltpu.async_copy(fgT_hbm.at[i0], frb.at[pl.ds(0, 96)], sem)
            pltpu.async_copy(fgT_hbm.at[i1], frb.at[pl.ds(96, 104)], sem)
            pltpu.async_copy(lbs_hbm.at[i0], lrb.at[pl.ds(0, 96)], sem)
            pltpu.async_copy(lbs_hbm.at[i1], lrb.at[pl.ds(96, 104)], sem)

        def drain(fqb, frb, lrb, sem):
            pltpu.make_async_copy(fq_hbm.at[pl.ds(0, _QC)], fqb, sem).wait()
            pltpu.make_async_copy(fgT_hbm.at[pl.ds(0, rows)], frb, sem).wait()
            pltpu.make_async_copy(lbs_hbm.at[pl.ds(0, rows)], lrb, sem).wait()

        def compute(ch, fqb, frb, lrb):
            def q_body(qi, _):
                fqc = [fqb[qi, pl.ds(16 * t, 16)] for t in range(2 * nb32)]

                def o_body(o, carry):
                    a0, a1, dsum = carry
                    row = o * _QC + qi
                    sa = [jnp.zeros((16,), f32) for _ in range(4)]
                    for t in range(nb32):
                        w32 = frb[row, pl.ds(16 * t, 16)]
                        ga = lax.bitcast_convert_type(
                            w32 << jnp.int32(16), f32)
                        gb = lax.bitcast_convert_type(
                            w32 & jnp.int32(-65536), f32)
                        d0 = ga - fqc[t]
                        d1 = gb - fqc[t + nb32]
                        sa[(2 * t) % 4] = sa[(2 * t) % 4] + d0 * d0
                        sa[(2 * t + 1) % 4] = sa[(2 * t + 1) % 4] + d1 * d1
                    s = (sa[0] + sa[1]) + (sa[2] + sa[3])
                    for p in perms:
                        s = s + lax.gather(
                            s, p[:, None],
                            lax.GatherDimensionNumbers(
                                offset_dims=(), collapsed_slice_dims=(0,),
                                start_index_map=(0,)),
                            slice_sizes=(1,),
                            mode=lax.GatherScatterMode.PROMISE_IN_BOUNDS)
                    e = jnp.exp(-(s * wreg))
                    l0 = lrb[row, pl.ds(0, 16)]
                    l1 = lrb[row, pl.ds(16, 16)]
                    return (a0 + l0 * e, a1 + l1 * e, dsum + e)

                z = jnp.zeros((16,), f32)
                a0, a1, dsum = lax.fori_loop(0, 25, o_body, (z, z, z))
                dm = jnp.maximum(dsum, 1e-15)
                orow = ch * _QC + qi
                outb_v[orow, pl.ds(0, 16)] = a0 / dm
                outb_v[orow, pl.ds(16, 16)] = a1 / dm
                return 0

            lax.fori_loop(0, _QC, q_body, 0)

        fire(0, fq_a, fr_a, lr_a, sem_a)

        def pair_body(i, _):
            c0 = 2 * i
            fire(c0 + 1, fq_b, fr_b, lr_b, sem_b)
            drain(fq_a, fr_a, lr_a, sem_a)
            compute(c0, fq_a, fr_a, lr_a)
            fire(c0 + 2, fq_a, fr_a, lr_a, sem_a)
            drain(fq_b, fr_b, lr_b, sem_b)
            compute(c0 + 1, fq_b, fr_b, lr_b)
            return 0

        lax.fori_loop(0, cpw // 2 - 1, pair_body, 0)
        fire(cpw - 1, fq_b, fr_b, lr_b, sem_b)
        drain(fq_a, fr_a, lr_a, sem_a)
        compute(cpw - 2, fq_a, fr_a, lr_a)
        drain(fq_b, fr_b, lr_b, sem_b)
        compute(cpw - 1, fq_b, fr_b, lr_b)
        pltpu.sync_copy(outb_v, out_hbm.at[pl.ds(w * qpw, qpw)])

    out = _sc(fgT3, featqT, lbsTp, idx_blocked, wsc)
    return (out.reshape(N, Q, 32)[:, :, :ncls]
            .transpose(0, 2, 1).reshape(N, ncls, r_in, c_in))


# speedup vs baseline: 3.3125x; 1.0346x over previous
import functools

import jax
import jax.numpy as jnp
import numpy as np
from jax import lax
from jax.experimental import pallas as pl
from jax.experimental.pallas import tpu as pltpu
from jax.experimental.pallas import tpu_sc as plsc

_QC = 8


def _mm_rows(x_ref, u_ref, y_ref):
    y_ref[0] = jnp.dot(u_ref[...], x_ref[0], preferred_element_type=jnp.float32)


def _make_mm_cols(D, gpb):
    def _mm_cols(y_ref, u_ref, o_ref):
        cat = jnp.concatenate([y_ref[0, j] for j in range(gpb)], axis=1)
        res = jnp.dot(u_ref[...], cat, preferred_element_type=jnp.float32)
        for j in range(gpb):
            rj = res[:, j * D:(j + 1) * D]
            lo = rj[:, :D // 2].astype(jnp.bfloat16)
            hi = rj[:, D // 2:].astype(jnp.bfloat16)
            lo32 = jax.lax.bitcast_convert_type(lo, jnp.uint16).astype(jnp.int32)
            hi32 = jax.lax.bitcast_convert_type(hi, jnp.uint16).astype(jnp.int32)
            o_ref[0, j] = lo32 | (hi32 << jnp.int32(16))

    return _mm_cols


def kernel(grid_inte, grid, feat, lbs, wei):
    f32 = jnp.float32
    N, D = feat.shape[0], feat.shape[1]
    r, c = grid.shape[1], grid.shape[2]
    r_in, c_in = grid_inte.shape[1], grid_inte.shape[2]
    ncls = lbs.shape[1]
    Q = r_in * c_in
    RC = r * c
    NQ = N * Q

    gX = grid[0, 2, 2, 0] - grid[0, 1, 2, 0]
    gY = grid[0, 2, 2, 1] - grid[0, 2, 1, 1]
    rp_x = jnp.floor(grid_inte[0, :, :, 0] / gX).reshape(-1)
    rp_y = jnp.floor(grid_inte[0, :, :, 1] / gY).reshape(-1)
    locs = []
    for i in range(-2, 3):
        for j in range(-2, 3):
            px = jnp.clip(rp_x + i, 0, r - 1)
            py = jnp.clip(rp_y + j, 0, c - 1)
            locs.append((px * c + py).astype(jnp.int32))
    loc_all = jnp.stack(locs, 0)
    idxg = (jnp.arange(N, dtype=jnp.int32)[None, :, None] * RC
            + loc_all[:, None, :])
    idxr = idxg.transpose(1, 2, 0).reshape(NQ, 25)
    idxp = jnp.pad(idxr, ((0, 0), (0, 32 - 25)))
    nch = NQ // _QC
    idx_blocked = idxp.reshape(nch, _QC, 32).transpose(0, 2, 1)
    idx_blocked = jnp.clip(idx_blocked.reshape(nch * 32 * _QC), 0, N * RC - 1)

    U = jax.image.resize(jnp.eye(c_in, dtype=f32), (c, c_in), method="bilinear")
    featT = feat.transpose(0, 2, 3, 1)
    Y = pl.pallas_call(
        _mm_rows,
        grid=(N,),
        in_specs=[
            pl.BlockSpec((1, r_in, c_in * D), lambda n: (n, 0, 0)),
            pl.BlockSpec((r, r_in), lambda n: (0, 0)),
        ],
        out_specs=pl.BlockSpec((1, r, c_in * D), lambda n: (n, 0, 0)),
        out_shape=jax.ShapeDtypeStruct((N, r, c_in * D), f32),
    )(featT.reshape(N, r_in, c_in * D), U)
    gpb = 8
    Y2 = Y.reshape(N * r // gpb, gpb, c_in, D)
    fgT = pl.pallas_call(
        _make_mm_cols(D, gpb),
        grid=(N * r // gpb,),
        in_specs=[
            pl.BlockSpec((1, gpb, c_in, D), lambda g: (g, 0, 0, 0)),
            pl.BlockSpec((c, c_in), lambda g: (0, 0)),
        ],
        out_specs=pl.BlockSpec((1, gpb, c, D // 2), lambda g: (g, 0, 0, 0)),
        out_shape=jax.ShapeDtypeStruct((N * r // gpb, gpb, c, D // 2), jnp.int32),
    )(Y2, U)
    fgT3 = fgT.reshape(N * r * c, D // 2)

    featqT = featT.reshape(NQ, D)
    lbsT = lbs.transpose(0, 2, 3, 1).reshape(N * RC, ncls)
    lbsTp = jnp.pad(lbsT, ((0, 0), (0, 32 - ncls)))
    llo = lax.bitcast_convert_type(
        lbsTp[:, :16].astype(jnp.bfloat16), jnp.uint16).astype(jnp.int32)
    lhi = lax.bitcast_convert_type(
        lbsTp[:, 16:].astype(jnp.bfloat16), jnp.uint16).astype(jnp.int32)
    lbsP = llo | (lhi << jnp.int32(16))
    wsc = jnp.broadcast_to((wei[0] * (1.0 / D)).astype(f32), (16,))

    info = plsc.get_sparse_core_info()
    nw = info.num_cores * info.num_subcores
    cpw = nch // nw
    qpw = cpw * _QC
    rows = 25 * _QC
    nb32 = D // 32
    mesh = plsc.VectorSubcoreMesh(core_axis_name="c", subcore_axis_name="s")

    @functools.partial(
        pl.kernel,
        out_type=jax.ShapeDtypeStruct((NQ, 32), f32),
        mesh=mesh,
        scratch_types=[
            pltpu.VMEM((cpw * 32 * _QC,), jnp.int32),
            pltpu.VMEM((qpw, 32), f32),
            pltpu.VMEM((16,), f32),
        ]
        + 2 * [
            pltpu.VMEM((_QC, D), f32),
            pltpu.VMEM((rows, D // 2), jnp.int32),
            pltpu.VMEM((rows, 16), jnp.int32),
            pltpu.SemaphoreType.DMA,
        ],
        compiler_params=pltpu.CompilerParams(use_tc_tiling_on_sc=False),
    )
    def _sc(fgT_hbm, fq_hbm, lbs_hbm, idx_hbm, wsc_hbm, out_hbm,
            idx_v, outb_v, wsc_v,
            fq_a, fr_a, lr_a, sem_a, fq_b, fr_b, lr_b, sem_b):
        w = lax.axis_index("s") * info.num_cores + lax.axis_index("c")
        pltpu.sync_copy(wsc_hbm, wsc_v)
        pltpu.sync_copy(idx_hbm.at[pl.ds(w * cpw * 256, cpw * 256)], idx_v)
        wreg = wsc_v[...]
        lane = lax.iota(jnp.int32, 16)
        perms = [(lane + k) & 15 for k in (8, 4, 2, 1)]

        def fire(ch, fqb, frb, lrb, sem):
            i0 = idx_v.at[pl.ds(ch * 256, 96)]
            i1 = idx_v.at[pl.ds(ch * 256 + 96, 104)]
            pltpu.async_copy(
                fq_hbm.at[pl.ds((w * cpw + ch) * _QC, _QC)], fqb, sem)
            pltpu.async_copy(fgT_hbm.at[i0], frb.at[pl.ds(0, 96)], sem)
            pltpu.async_copy(fgT_hbm.at[i1], frb.at[pl.ds(96, 104)], sem)
            pltpu.async_copy(lbs_hbm.at[i0], lrb.at[pl.ds(0, 96)], sem)
            pltpu.async_copy(lbs_hbm.at[i1], lrb.at[pl.ds(96, 104)], sem)

        def drain(fqb, frb, lrb, sem):
            pltpu.make_async_copy(fq_hbm.at[pl.ds(0, _QC)], fqb, sem).wait()
            pltpu.make_async_copy(fgT_hbm.at[pl.ds(0, rows)], frb, sem).wait()
            pltpu.make_async_copy(lbs_hbm.at[pl.ds(0, rows)], lrb, sem).wait()

        def lane_shuffle(s, p):
            return lax.gather(
                s, p[:, None],
                lax.GatherDimensionNumbers(
                    offset_dims=(), collapsed_slice_dims=(0,),
                    start_index_map=(0,)),
                slice_sizes=(1,),
                mode=lax.GatherScatterMode.PROMISE_IN_BOUNDS)

        def compute(ch, fqb, frb, lrb):
            def q_body(qi, _):
                fqc = [fqb[qi, pl.ds(16 * t, 16)] for t in range(2 * nb32)]

                def g_body(g, carry):
                    a0, a1, dsum = carry
                    rws = [(g * 5 + k) * _QC + qi for k in range(5)]
                    svals = []
                    for row in rws:
                        sa = [jnp.zeros((16,), f32) for _ in range(4)]
                        for t in range(nb32):
                            w32 = frb[row, pl.ds(16 * t, 16)]
                            ga = lax.bitcast_convert_type(
                                w32 << jnp.int32(16), f32)
                            gb = lax.bitcast_convert_type(
                                w32 & jnp.int32(-65536), f32)
                            d0 = ga - fqc[t]
                            d1 = gb - fqc[t + nb32]
                            sa[(2 * t) % 4] = sa[(2 * t) % 4] + d0 * d0
                            sa[(2 * t + 1) % 4] = sa[(2 * t + 1) % 4] + d1 * d1
                        svals.append((sa[0] + sa[1]) + (sa[2] + sa[3]))
                    for p in perms:
                        svals = [s + lane_shuffle(s, p) for s in svals]
                    es = [jnp.exp(-(s * wreg)) for s in svals]
                    lws = [lrb[row, :] for row in rws]
                    v0 = [lax.bitcast_convert_type(lw << jnp.int32(16), f32) * e
                          for lw, e in zip(lws, es)]
                    v1 = [lax.bitcast_convert_type(lw & jnp.int32(-65536), f32) * e
                          for lw, e in zip(lws, es)]
                    a0 = a0 + (((v0[0] + v0[1]) + (v0[2] + v0[3])) + v0[4])
                    a1 = a1 + (((v1[0] + v1[1]) + (v1[2] + v1[3])) + v1[4])
                    dsum = dsum + (((es[0] + es[1]) + (es[2] + es[3])) + es[4])
                    return (a0, a1, dsum)

                z = jnp.zeros((16,), f32)
                a0, a1, dsum = lax.fori_loop(0, 5, g_body, (z, z, z))
                dm = jnp.maximum(dsum, 1e-15)
                orow = ch * _QC + qi
                outb_v[orow, pl.ds(0, 16)] = a0 / dm
                outb_v[orow, pl.ds(16, 16)] = a1 / dm
                return 0

            lax.fori_loop(0, _QC, q_body, 0)

        fire(0, fq_a, fr_a, lr_a, sem_a)

        def pair_body(i, _):
            c0 = 2 * i
            fire(c0 + 1, fq_b, fr_b, lr_b, sem_b)
            drain(fq_a, fr_a, lr_a, sem_a)
            compute(c0, fq_a, fr_a, lr_a)
            fire(c0 + 2, fq_a, fr_a, lr_a, sem_a)
            drain(fq_b, fr_b, lr_b, sem_b)
            compute(c0 + 1, fq_b, fr_b, lr_b)
            return 0

        lax.fori_loop(0, cpw // 2 - 1, pair_body, 0)
        fire(cpw - 1, fq_b, fr_b, lr_b, sem_b)
        drain(fq_a, fr_a, lr_a, sem_a)
        compute(cpw - 2, fq_a, fr_a, lr_a)
        drain(fq_b, fr_b, lr_b, sem_b)
        compute(cpw - 1, fq_b, fr_b, lr_b)
        pltpu.sync_copy(outb_v, out_hbm.at[pl.ds(w * qpw, qpw)])

    out = _sc(fgT3, featqT, lbsP, idx_blocked, wsc)
    return (out.reshape(N, Q, 32)[:, :, :ncls]
            .transpose(0, 2, 1).reshape(N, ncls, r_in, c_in))
